# Initial kernel scaffold; baseline (speedup 1.0000x reference)
import jax
import jax.numpy as jnp
from jax.experimental import pallas as pl


def kernel(x, edge_index, W1_0, b1_0, g_0, be_0, m_0, v_0, W2_0, b2_0, W1_1, b1_1, g_1, be_1, m_1, v_1, W2_1, b2_1, W1_2, b1_2, g_2, be_2, m_2, v_2, W2_2, b2_2, lin1_W, lin1_b, lin2_W, lin2_b):
    raise NotImplementedError("write your pallas kernel here")


# speedup vs baseline: 4.3939x; 4.3939x over previous
import functools

import jax
import jax.numpy as jnp
from jax import lax
from jax.experimental import pallas as pl
from jax.experimental.pallas import tpu as pltpu
from jax.experimental.pallas import tpu_sc as plsc

N = 10000
E = 320000
D = 128

NC = 2
NS = 16
NW = NC * NS

EDGES_PER_TILE = E // NW
CHUNK = 80
NUM_CHUNKS = EDGES_PER_TILE // CHUNK
NPAD = 12800
ROWS_PER_TILE = NPAD // NS
ZERO_ROWS = 32


def _sc_aggregate_body(x_hbm, src_hbm, dst_hbm, out_hbm,
                       src_v, dst_v, rows_v, zero_v, acc_sh, sem):
    cid = lax.axis_index("c")
    sid = lax.axis_index("s")
    wid = cid * NS + sid

    for r in range(ZERO_ROWS):
        for c in range(D // 16):
            zero_v[r, pl.ds(c * 16, 16)] = jnp.zeros((16,), jnp.float32)
    row0 = sid * ROWS_PER_TILE

    def _zero_body(j):
        pltpu.sync_copy(zero_v, acc_sh.at[pl.ds(row0 + j * ZERO_ROWS, ZERO_ROWS)])
    pl.loop(0, ROWS_PER_TILE // ZERO_ROWS)(_zero_body)

    plsc.subcore_barrier()

    base = wid * EDGES_PER_TILE

    def _edge_body(i):
        off = base + i * CHUNK
        pltpu.sync_copy(src_hbm.at[pl.ds(off, CHUNK)], src_v)
        pltpu.sync_copy(dst_hbm.at[pl.ds(off, CHUNK)], dst_v)
        pltpu.async_copy(x_hbm.at[src_v], rows_v, sem).wait()
        pltpu.sync_copy(rows_v, acc_sh.at[dst_v], add=True)
    pl.loop(0, NUM_CHUNKS)(_edge_body)

    plsc.subcore_barrier()

    pltpu.sync_copy(acc_sh.at[pl.ds(row0, ROWS_PER_TILE)],
                    out_hbm.at[pl.ds(cid * NPAD + row0, ROWS_PER_TILE)])


@functools.lru_cache(maxsize=1)
def _build_sc_aggregate():
    mesh = plsc.VectorSubcoreMesh(core_axis_name="c", subcore_axis_name="s",
                                  num_cores=NC, num_subcores=NS)
    return pl.kernel(
        _sc_aggregate_body,
        out_type=jax.ShapeDtypeStruct((NC * NPAD, D), jnp.float32),
        mesh=mesh,
        scratch_types=[
            pltpu.VMEM((CHUNK,), jnp.int32),
            pltpu.VMEM((CHUNK,), jnp.int32),
            pltpu.VMEM((CHUNK, D), jnp.float32),
            pltpu.VMEM((ZERO_ROWS, D), jnp.float32),
            pltpu.VMEM_SHARED((NPAD, D), jnp.float32),
            pltpu.SemaphoreType.DMA,
        ],
    )


def _sc_aggregate(x, src, dst):
    return _build_sc_aggregate()(x, src, dst)


BLK = 400


def _mlp_body(x_ref, p0_ref, p1_ref, w1_ref, b1_ref, w2_ref, b2_ref, o_ref):
    h = x_ref[...] + p0_ref[...] + p1_ref[...]
    h = jnp.dot(h, w1_ref[...], preferred_element_type=jnp.float32) + b1_ref[...]
    h = jnp.maximum(h, 0.0)
    h = jnp.dot(h, w2_ref[...], preferred_element_type=jnp.float32) + b2_ref[...]
    o_ref[...] = jnp.maximum(h, 0.0)


def _head_body(x_ref, p0_ref, p1_ref, w1_ref, b1_ref, w2_ref, b2_ref,
               l1w_ref, l1b_ref, l2w_ref, l2b_ref, o_ref):
    h = x_ref[...] + p0_ref[...] + p1_ref[...]
    h = jnp.dot(h, w1_ref[...], preferred_element_type=jnp.float32) + b1_ref[...]
    h = jnp.maximum(h, 0.0)
    h = jnp.dot(h, w2_ref[...], preferred_element_type=jnp.float32) + b2_ref[...]
    h = jnp.maximum(h, 0.0)
    h = jnp.dot(h, l1w_ref[...], preferred_element_type=jnp.float32) + l1b_ref[...]
    h = jnp.maximum(h, 0.0)
    l = jnp.dot(h, l2w_ref[...], preferred_element_type=jnp.float32) + l2b_ref[...]
    m = jnp.max(l, axis=-1, keepdims=True)
    lse = jnp.log(jnp.sum(jnp.exp(l - m), axis=-1, keepdims=True)) + m
    o_ref[...] = l - lse


def _row_spec():
    return pl.BlockSpec((BLK, D), lambda i: (i, 0))


def _full_spec(shape):
    return pl.BlockSpec(shape, lambda i: tuple(0 for _ in shape))


def _mlp(x, parts, w1, b1, w2, b2):
    return pl.pallas_call(
        _mlp_body,
        grid=(N // BLK,),
        in_specs=[
            _row_spec(),
            pl.BlockSpec((BLK, D), lambda i: (i, 0)),
            pl.BlockSpec((BLK, D), lambda i: (i + NPAD // BLK, 0)),
            _full_spec((D, D)), _full_spec((1, D)),
            _full_spec((D, D)), _full_spec((1, D)),
        ],
        out_specs=_row_spec(),
        out_shape=jax.ShapeDtypeStruct((N, D), jnp.float32),
    )(x, parts, parts, w1, b1, w2, b2)


def _head(x, parts, w1, b1, w2, b2, l1w, l1b, l2w, l2b, C):
    return pl.pallas_call(
        _head_body,
        grid=(N // BLK,),
        in_specs=[
            _row_spec(),
            pl.BlockSpec((BLK, D), lambda i: (i, 0)),
            pl.BlockSpec((BLK, D), lambda i: (i + NPAD // BLK, 0)),
            _full_spec((D, D)), _full_spec((1, D)),
            _full_spec((D, D)), _full_spec((1, D)),
            _full_spec((D, D)), _full_spec((1, D)),
            _full_spec((D, C)), _full_spec((1, C)),
        ],
        out_specs=pl.BlockSpec((BLK, C), lambda i: (i, 0)),
        out_shape=jax.ShapeDtypeStruct((N, C), jnp.float32),
    )(x, parts, parts, w1, b1, w2, b2, l1w, l1b, l2w, l2b)


def _fold_bn(W1, b1, g, be, m, v):
    s = g / jnp.sqrt(v + 1e-5)
    return W1 * s[None, :], ((b1 - m) * s + be)[None, :]


def kernel(x, edge_index, W1_0, b1_0, g_0, be_0, m_0, v_0, W2_0, b2_0,
           W1_1, b1_1, g_1, be_1, m_1, v_1, W2_1, b2_1,
           W1_2, b1_2, g_2, be_2, m_2, v_2, W2_2, b2_2,
           lin1_W, lin1_b, lin2_W, lin2_b):
    src = edge_index[0]
    dst = edge_index[1]
    C = lin2_W.shape[1]

    w1f_0, b1f_0 = _fold_bn(W1_0, b1_0, g_0, be_0, m_0, v_0)
    w1f_1, b1f_1 = _fold_bn(W1_1, b1_1, g_1, be_1, m_1, v_1)
    w1f_2, b1f_2 = _fold_bn(W1_2, b1_2, g_2, be_2, m_2, v_2)

    p = _sc_aggregate(x, src, dst)
    h = _mlp(x, p, w1f_0, b1f_0, W2_0, b2_0[None, :])
    p = _sc_aggregate(h, src, dst)
    h = _mlp(h, p, w1f_1, b1f_1, W2_1, b2_1[None, :])
    p = _sc_aggregate(h, src, dst)
    return _head(h, p, w1f_2, b1f_2, W2_2, b2_2[None, :],
                 lin1_W, lin1_b[None, :], lin2_W, lin2_b[None, :], C)


# speedup vs baseline: 7.6877x; 1.7496x over previous
import functools

import jax
import jax.numpy as jnp
from jax import lax
from jax.experimental import pallas as pl
from jax.experimental.pallas import tpu as pltpu
from jax.experimental.pallas import tpu_sc as plsc

N = 10000
E = 320000
D = 128

NC = 2
NS = 16
NW = NC * NS

EDGES_PER_TILE = E // NW
CHUNK = 80
NUM_CHUNKS = EDGES_PER_TILE // CHUNK
NSTAGE = 5
STAGE = NUM_CHUNKS // NSTAGE
NPAD = 10112
ROWS_PER_TILE = NPAD // NS


def _sc_aggregate_body(x_hbm, src_hbm, dst_hbm, out_hbm,
                       si, di, r0, r1, acc_sh, gsem, ssem, zsem):
    cid = lax.axis_index("c")
    sid = lax.axis_index("s")
    wid = cid * NS + sid
    rows = [r0, r1]

    for r in range(CHUNK):
        for c in range(D // 16):
            r0[r, pl.ds(c * 16, 16)] = jnp.zeros((16,), jnp.float32)
    row0 = sid * ROWS_PER_TILE
    zdescs = [
        pltpu.async_copy(r0, acc_sh.at[pl.ds(row0 + j * CHUNK, CHUNK)], zsem)
        for j in range(ROWS_PER_TILE // CHUNK)
    ]
    zrem = ROWS_PER_TILE % CHUNK
    if zrem:
        zdescs.append(pltpu.async_copy(
            r0.at[pl.ds(0, zrem)],
            acc_sh.at[pl.ds(row0 + ROWS_PER_TILE - zrem, zrem)], zsem))
    for d in zdescs:
        d.wait()

    plsc.subcore_barrier()

    def g_start(lc, k):
        pltpu.async_copy(x_hbm.at[si.at[lc]], rows[k], gsem.at[k])

    def g_wait(k):
        pltpu.make_async_copy(x_hbm.at[si.at[0]], rows[k], gsem.at[k]).wait()

    def s_start(lc, k):
        pltpu.async_copy(rows[k], acc_sh.at[di.at[lc]], ssem.at[k], add=True)

    def s_wait(k):
        pltpu.make_async_copy(rows[k], acc_sh.at[di.at[0]], ssem.at[k]).wait()

    for s in range(NSTAGE):
        b0 = s % 2
        b1 = 1 - b0
        pltpu.sync_copy(src_hbm.at[wid, s], si)
        pltpu.sync_copy(dst_hbm.at[wid, s], di)
        g_start(0, b0)
        g_start(1, b1)
        g_wait(b0)
        s_start(0, b0)
        s_wait(b0)
        g_start(2, b0)
        g_wait(b1)
        s_start(1, b1)

        def duo(j, b0=b0, b1=b1):
            s_wait(b1)
            g_start(j + 1, b1)
            g_wait(b0)
            s_start(j, b0)
            s_wait(b0)
            g_start(j + 2, b0)
            g_wait(b1)
            s_start(j + 1, b1)
        pl.loop(2, STAGE - 1, step=2)(duo)
        g_wait(b0)
        s_start(STAGE - 1, b0)
        s_wait(b0)
        s_wait(b1)

    plsc.subcore_barrier()

    pltpu.sync_copy(acc_sh.at[pl.ds(row0, ROWS_PER_TILE)],
                    out_hbm.at[pl.ds(cid * NPAD + row0, ROWS_PER_TILE)])


@functools.lru_cache(maxsize=1)
def _build_sc_aggregate():
    mesh = plsc.VectorSubcoreMesh(core_axis_name="c", subcore_axis_name="s",
                                  num_cores=NC, num_subcores=NS)
    return pl.kernel(
        _sc_aggregate_body,
        out_type=jax.ShapeDtypeStruct((NC * NPAD, D), jnp.float32),
        mesh=mesh,
        scratch_types=[
            pltpu.VMEM((STAGE, CHUNK), jnp.int32),
            pltpu.VMEM((STAGE, CHUNK), jnp.int32),
            pltpu.VMEM((CHUNK, D), jnp.float32),
            pltpu.VMEM((CHUNK, D), jnp.float32),
            pltpu.VMEM_SHARED((NPAD, D), jnp.float32),
            pltpu.SemaphoreType.DMA((2,)),
            pltpu.SemaphoreType.DMA((2,)),
            pltpu.SemaphoreType.DMA,
        ],
    )


def _sc_aggregate(x, src, dst):
    src4 = src.reshape(NW, NSTAGE, STAGE, CHUNK)
    dst4 = dst.reshape(NW, NSTAGE, STAGE, CHUNK)
    return _build_sc_aggregate()(x, src4, dst4)


BLK = 128
GRID = (N + BLK - 1) // BLK


def _mlp_body(x_ref, p0_ref, p1_ref, w1_ref, b1_ref, w2_ref, b2_ref, o_ref):
    h = x_ref[...] + p0_ref[...] + p1_ref[...]
    h = jnp.dot(h, w1_ref[...], preferred_element_type=jnp.float32) + b1_ref[...]
    h = jnp.maximum(h, 0.0)
    h = jnp.dot(h, w2_ref[...], preferred_element_type=jnp.float32) + b2_ref[...]
    o_ref[...] = jnp.maximum(h, 0.0)


def _head_body(x_ref, p0_ref, p1_ref, w1_ref, b1_ref, w2_ref, b2_ref,
               l1w_ref, l1b_ref, l2w_ref, l2b_ref, o_ref):
    h = x_ref[...] + p0_ref[...] + p1_ref[...]
    h = jnp.dot(h, w1_ref[...], preferred_element_type=jnp.float32) + b1_ref[...]
    h = jnp.maximum(h, 0.0)
    h = jnp.dot(h, w2_ref[...], preferred_element_type=jnp.float32) + b2_ref[...]
    h = jnp.maximum(h, 0.0)
    h = jnp.dot(h, l1w_ref[...], preferred_element_type=jnp.float32) + l1b_ref[...]
    h = jnp.maximum(h, 0.0)
    l = jnp.dot(h, l2w_ref[...], preferred_element_type=jnp.float32) + l2b_ref[...]
    m = jnp.max(l, axis=-1, keepdims=True)
    lse = jnp.log(jnp.sum(jnp.exp(l - m), axis=-1, keepdims=True)) + m
    o_ref[...] = l - lse


def _row_spec():
    return pl.BlockSpec((BLK, D), lambda i: (i, 0))


def _full_spec(shape):
    return pl.BlockSpec(shape, lambda i: tuple(0 for _ in shape))


def _mlp(x, parts, w1, b1, w2, b2):
    return pl.pallas_call(
        _mlp_body,
        grid=(GRID,),
        in_specs=[
            _row_spec(),
            pl.BlockSpec((BLK, D), lambda i: (i, 0)),
            pl.BlockSpec((BLK, D), lambda i: (i + NPAD // BLK, 0)),
            _full_spec((D, D)), _full_spec((1, D)),
            _full_spec((D, D)), _full_spec((1, D)),
        ],
        out_specs=_row_spec(),
        out_shape=jax.ShapeDtypeStruct((N, D), jnp.float32),
    )(x, parts, parts, w1, b1, w2, b2)


def _head(x, parts, w1, b1, w2, b2, l1w, l1b, l2w, l2b, C):
    return pl.pallas_call(
        _head_body,
        grid=(GRID,),
        in_specs=[
            _row_spec(),
            pl.BlockSpec((BLK, D), lambda i: (i, 0)),
            pl.BlockSpec((BLK, D), lambda i: (i + NPAD // BLK, 0)),
            _full_spec((D, D)), _full_spec((1, D)),
            _full_spec((D, D)), _full_spec((1, D)),
            _full_spec((D, D)), _full_spec((1, D)),
            _full_spec((D, C)), _full_spec((1, C)),
        ],
        out_specs=pl.BlockSpec((BLK, C), lambda i: (i, 0)),
        out_shape=jax.ShapeDtypeStruct((N, C), jnp.float32),
    )(x, parts, parts, w1, b1, w2, b2, l1w, l1b, l2w, l2b)


def _fold_bn(W1, b1, g, be, m, v):
    s = g / jnp.sqrt(v + 1e-5)
    return W1 * s[None, :], ((b1 - m) * s + be)[None, :]


def kernel(x, edge_index, W1_0, b1_0, g_0, be_0, m_0, v_0, W2_0, b2_0,
           W1_1, b1_1, g_1, be_1, m_1, v_1, W2_1, b2_1,
           W1_2, b1_2, g_2, be_2, m_2, v_2, W2_2, b2_2,
           lin1_W, lin1_b, lin2_W, lin2_b):
    src = edge_index[0]
    dst = edge_index[1]
    C = lin2_W.shape[1]

    w1f_0, b1f_0 = _fold_bn(W1_0, b1_0, g_0, be_0, m_0, v_0)
    w1f_1, b1f_1 = _fold_bn(W1_1, b1_1, g_1, be_1, m_1, v_1)
    w1f_2, b1f_2 = _fold_bn(W1_2, b1_2, g_2, be_2, m_2, v_2)

    p = _sc_aggregate(x, src, dst)
    h = _mlp(x, p, w1f_0, b1f_0, W2_0, b2_0[None, :])
    p = _sc_aggregate(h, src, dst)
    h = _mlp(h, p, w1f_1, b1f_1, W2_1, b2_1[None, :])
    p = _sc_aggregate(h, src, dst)
    return _head(h, p, w1f_2, b1f_2, W2_2, b2_2[None, :],
                 lin1_W, lin1_b[None, :], lin2_W, lin2_b[None, :], C)


# speedup vs baseline: 8.1003x; 1.0537x over previous
import functools

import jax
import jax.numpy as jnp
from jax import lax
from jax.experimental import pallas as pl
from jax.experimental.pallas import tpu as pltpu
from jax.experimental.pallas import tpu_sc as plsc

N = 10000
E = 320000
D = 128

NC = 2
NS = 16
NW = NC * NS

EDGES_PER_TILE = E // NW
CHUNK = 40
NUM_CHUNKS = EDGES_PER_TILE // CHUNK
NSTAGE = 5
STAGE = NUM_CHUNKS // NSTAGE
NBUF = 4
LOOK = 2
NPAD = 10112
ROWS_PER_TILE = NPAD // NS


def _sc_aggregate_body(x_hbm, src_hbm, dst_hbm, out_hbm,
                       si, di, r0, r1, r2, r3, acc_sh, gsem, ssem, zsem):
    cid = lax.axis_index("c")
    sid = lax.axis_index("s")
    wid = cid * NS + sid
    rows = [r0, r1, r2, r3]

    for r in range(CHUNK):
        for c in range(D // 16):
            r0[r, pl.ds(c * 16, 16)] = jnp.zeros((16,), jnp.float32)
    row0 = sid * ROWS_PER_TILE
    zdescs = [
        pltpu.async_copy(r0, acc_sh.at[pl.ds(row0 + j * CHUNK, CHUNK)], zsem)
        for j in range(ROWS_PER_TILE // CHUNK)
    ]
    zrem = ROWS_PER_TILE % CHUNK
    if zrem:
        zdescs.append(pltpu.async_copy(
            r0.at[pl.ds(0, zrem)],
            acc_sh.at[pl.ds(row0 + ROWS_PER_TILE - zrem, zrem)], zsem))
    for d in zdescs:
        d.wait()

    plsc.subcore_barrier()

    def g_start(lc, k):
        pltpu.async_copy(x_hbm.at[si.at[lc]], rows[k], gsem.at[k])

    def g_wait(k):
        pltpu.make_async_copy(x_hbm.at[si.at[0]], rows[k], gsem.at[k]).wait()

    def s_start(lc, k):
        pltpu.async_copy(rows[k], acc_sh.at[di.at[lc]], ssem.at[k], add=True)

    def s_wait(k):
        pltpu.make_async_copy(rows[k], acc_sh.at[di.at[0]], ssem.at[k]).wait()

    for s in range(NSTAGE):
        base = (2 * s) % NBUF

        def B(x, base=base):
            return (x + base) % NBUF

        pltpu.sync_copy(src_hbm.at[wid, s], si)
        pltpu.sync_copy(dst_hbm.at[wid, s], di)
        g_start(0, B(0))
        g_start(1, B(1))
        g_start(2, B(2))
        g_wait(B(0))
        s_start(0, B(0))
        g_start(3, B(3))
        g_wait(B(1))
        s_start(1, B(1))

        def quad(j, base=base):
            for k in range(NBUF):
                c = j + k
                s_wait((k + base) % NBUF)
                g_start(c + LOOK, (k + base) % NBUF)
                g_wait((2 + k + base) % NBUF)
                s_start(c, (2 + k + base) % NBUF)
        pl.loop(2, STAGE - NBUF, step=NBUF)(quad)
        for k in range(2):
            c = STAGE - NBUF + k
            s_wait((k + base) % NBUF)
            g_start(c + LOOK, (k + base) % NBUF)
            g_wait((2 + k + base) % NBUF)
            s_start(c, (2 + k + base) % NBUF)
        g_wait(B(STAGE - 2))
        s_start(STAGE - 2, B(STAGE - 2))
        g_wait(B(STAGE - 1))
        s_start(STAGE - 1, B(STAGE - 1))
        for k in range(NBUF):
            s_wait(k)

    plsc.subcore_barrier()

    pltpu.sync_copy(acc_sh.at[pl.ds(row0, ROWS_PER_TILE)],
                    out_hbm.at[pl.ds(cid * NPAD + row0, ROWS_PER_TILE)])


@functools.lru_cache(maxsize=1)
def _build_sc_aggregate():
    mesh = plsc.VectorSubcoreMesh(core_axis_name="c", subcore_axis_name="s",
                                  num_cores=NC, num_subcores=NS)
    return pl.kernel(
        _sc_aggregate_body,
        out_type=jax.ShapeDtypeStruct((NC * NPAD, D), jnp.float32),
        mesh=mesh,
        scratch_types=[
            pltpu.VMEM((STAGE, CHUNK), jnp.int32),
            pltpu.VMEM((STAGE, CHUNK), jnp.int32),
            pltpu.VMEM((CHUNK, D), jnp.float32),
            pltpu.VMEM((CHUNK, D), jnp.float32),
            pltpu.VMEM((CHUNK, D), jnp.float32),
            pltpu.VMEM((CHUNK, D), jnp.float32),
            pltpu.VMEM_SHARED((NPAD, D), jnp.float32),
            pltpu.SemaphoreType.DMA((NBUF,)),
            pltpu.SemaphoreType.DMA((NBUF,)),
            pltpu.SemaphoreType.DMA,
        ],
    )


def _sc_aggregate(x, src, dst):
    src4 = src.reshape(NW, NSTAGE, STAGE, CHUNK)
    dst4 = dst.reshape(NW, NSTAGE, STAGE, CHUNK)
    return _build_sc_aggregate()(x, src4, dst4)


BLK = 128
GRID = (N + BLK - 1) // BLK


def _mlp_body(x_ref, p0_ref, p1_ref, w1_ref, b1_ref, w2_ref, b2_ref, o_ref):
    h = x_ref[...] + p0_ref[...] + p1_ref[...]
    h = jnp.dot(h, w1_ref[...], preferred_element_type=jnp.float32) + b1_ref[...]
    h = jnp.maximum(h, 0.0)
    h = jnp.dot(h, w2_ref[...], preferred_element_type=jnp.float32) + b2_ref[...]
    o_ref[...] = jnp.maximum(h, 0.0)


def _head_body(x_ref, p0_ref, p1_ref, w1_ref, b1_ref, w2_ref, b2_ref,
               l1w_ref, l1b_ref, l2w_ref, l2b_ref, o_ref):
    h = x_ref[...] + p0_ref[...] + p1_ref[...]
    h = jnp.dot(h, w1_ref[...], preferred_element_type=jnp.float32) + b1_ref[...]
    h = jnp.maximum(h, 0.0)
    h = jnp.dot(h, w2_ref[...], preferred_element_type=jnp.float32) + b2_ref[...]
    h = jnp.maximum(h, 0.0)
    h = jnp.dot(h, l1w_ref[...], preferred_element_type=jnp.float32) + l1b_ref[...]
    h = jnp.maximum(h, 0.0)
    l = jnp.dot(h, l2w_ref[...], preferred_element_type=jnp.float32) + l2b_ref[...]
    m = jnp.max(l, axis=-1, keepdims=True)
    lse = jnp.log(jnp.sum(jnp.exp(l - m), axis=-1, keepdims=True)) + m
    o_ref[...] = l - lse


def _row_spec():
    return pl.BlockSpec((BLK, D), lambda i: (i, 0))


def _full_spec(shape):
    return pl.BlockSpec(shape, lambda i: tuple(0 for _ in shape))


def _mlp(x, parts, w1, b1, w2, b2):
    return pl.pallas_call(
        _mlp_body,
        grid=(GRID,),
        in_specs=[
            _row_spec(),
            pl.BlockSpec((BLK, D), lambda i: (i, 0)),
            pl.BlockSpec((BLK, D), lambda i: (i + NPAD // BLK, 0)),
            _full_spec((D, D)), _full_spec((1, D)),
            _full_spec((D, D)), _full_spec((1, D)),
        ],
        out_specs=_row_spec(),
        out_shape=jax.ShapeDtypeStruct((N, D), jnp.float32),
    )(x, parts, parts, w1, b1, w2, b2)


def _head(x, parts, w1, b1, w2, b2, l1w, l1b, l2w, l2b, C):
    return pl.pallas_call(
        _head_body,
        grid=(GRID,),
        in_specs=[
            _row_spec(),
            pl.BlockSpec((BLK, D), lambda i: (i, 0)),
            pl.BlockSpec((BLK, D), lambda i: (i + NPAD // BLK, 0)),
            _full_spec((D, D)), _full_spec((1, D)),
            _full_spec((D, D)), _full_spec((1, D)),
            _full_spec((D, D)), _full_spec((1, D)),
            _full_spec((D, C)), _full_spec((1, C)),
        ],
        out_specs=pl.BlockSpec((BLK, C), lambda i: (i, 0)),
        out_shape=jax.ShapeDtypeStruct((N, C), jnp.float32),
    )(x, parts, parts, w1, b1, w2, b2, l1w, l1b, l2w, l2b)


def _fold_bn(W1, b1, g, be, m, v):
    s = g / jnp.sqrt(v + 1e-5)
    return W1 * s[None, :], ((b1 - m) * s + be)[None, :]


def kernel(x, edge_index, W1_0, b1_0, g_0, be_0, m_0, v_0, W2_0, b2_0,
           W1_1, b1_1, g_1, be_1, m_1, v_1, W2_1, b2_1,
           W1_2, b1_2, g_2, be_2, m_2, v_2, W2_2, b2_2,
           lin1_W, lin1_b, lin2_W, lin2_b):
    src = edge_index[0]
    dst = edge_index[1]
    C = lin2_W.shape[1]

    w1f_0, b1f_0 = _fold_bn(W1_0, b1_0, g_0, be_0, m_0, v_0)
    w1f_1, b1f_1 = _fold_bn(W1_1, b1_1, g_1, be_1, m_1, v_1)
    w1f_2, b1f_2 = _fold_bn(W1_2, b1_2, g_2, be_2, m_2, v_2)

    p = _sc_aggregate(x, src, dst)
    h = _mlp(x, p, w1f_0, b1f_0, W2_0, b2_0[None, :])
    p = _sc_aggregate(h, src, dst)
    h = _mlp(h, p, w1f_1, b1f_1, W2_1, b2_1[None, :])
    p = _sc_aggregate(h, src, dst)
    return _head(h, p, w1f_2, b1f_2, W2_2, b2_2[None, :],
                 lin1_W, lin1_b[None, :], lin2_W, lin2_b[None, :], C)


# speedup vs baseline: 10.5264x; 1.2995x over previous
import functools

import jax
import jax.numpy as jnp
from jax import lax
from jax.experimental import pallas as pl
from jax.experimental.pallas import tpu as pltpu
from jax.experimental.pallas import tpu_sc as plsc

N = 10000
E = 320000
D = 128

NC = 2
NS = 16
NW = NC * NS

EDGES_PER_TILE = E // NW
CHUNK = 40
NUM_CHUNKS = EDGES_PER_TILE // CHUNK
NSTAGE = 5
STAGE = NUM_CHUNKS // NSTAGE
NBUF = 4
LOOK = 2
NPAD = 10112
ROWS_PER_TILE = NPAD // NS


def _sc_aggregate_body(x_hbm, ei_hbm, out_hbm,
                       si, di, r0, r1, r2, r3, acc_sh, gsem, ssem, zsem):
    cid = lax.axis_index("c")
    sid = lax.axis_index("s")
    wid = cid * NS + sid
    rows = [r0, r1, r2, r3]

    for r in range(CHUNK):
        for c in range(D // 16):
            r0[r, pl.ds(c * 16, 16)] = jnp.zeros((16,), jnp.float32)
    row0 = sid * ROWS_PER_TILE
    zdescs = [
        pltpu.async_copy(r0, acc_sh.at[pl.ds(row0 + j * CHUNK, CHUNK)], zsem)
        for j in range(ROWS_PER_TILE // CHUNK)
    ]
    zrem = ROWS_PER_TILE % CHUNK
    if zrem:
        zdescs.append(pltpu.async_copy(
            r0.at[pl.ds(0, zrem)],
            acc_sh.at[pl.ds(row0 + ROWS_PER_TILE - zrem, zrem)], zsem))
    for d in zdescs:
        d.wait()

    plsc.subcore_barrier()

    def g_start(lc, k):
        pltpu.async_copy(x_hbm.at[si.at[lc]], rows[k], gsem.at[k])

    def g_wait(k):
        pltpu.make_async_copy(x_hbm.at[si.at[0]], rows[k], gsem.at[k]).wait()

    def s_start(lc, k):
        pltpu.async_copy(rows[k], acc_sh.at[di.at[lc]], ssem.at[k], add=True)

    def s_wait(k):
        pltpu.make_async_copy(rows[k], acc_sh.at[di.at[0]], ssem.at[k]).wait()

    for s in range(NSTAGE):
        base = (2 * s) % NBUF

        def B(x, base=base):
            return (x + base) % NBUF

        pltpu.sync_copy(ei_hbm.at[0, wid, s], si)
        pltpu.sync_copy(ei_hbm.at[1, wid, s], di)
        g_start(0, B(0))
        g_start(1, B(1))
        g_start(2, B(2))
        g_wait(B(0))
        s_start(0, B(0))
        g_start(3, B(3))
        g_wait(B(1))
        s_start(1, B(1))

        def quad(j, base=base):
            for k in range(NBUF):
                c = j + k
                s_wait((k + base) % NBUF)
                g_start(c + LOOK, (k + base) % NBUF)
                g_wait((2 + k + base) % NBUF)
                s_start(c, (2 + k + base) % NBUF)
        pl.loop(2, STAGE - NBUF, step=NBUF)(quad)
        for k in range(2):
            c = STAGE - NBUF + k
            s_wait((k + base) % NBUF)
            g_start(c + LOOK, (k + base) % NBUF)
            g_wait((2 + k + base) % NBUF)
            s_start(c, (2 + k + base) % NBUF)
        g_wait(B(STAGE - 2))
        s_start(STAGE - 2, B(STAGE - 2))
        g_wait(B(STAGE - 1))
        s_start(STAGE - 1, B(STAGE - 1))
        for k in range(NBUF):
            s_wait(k)

    plsc.subcore_barrier()

    pltpu.sync_copy(acc_sh.at[pl.ds(row0, ROWS_PER_TILE)],
                    out_hbm.at[pl.ds(cid * NPAD + row0, ROWS_PER_TILE)])


@functools.lru_cache(maxsize=1)
def _build_sc_aggregate():
    mesh = plsc.VectorSubcoreMesh(core_axis_name="c", subcore_axis_name="s",
                                  num_cores=NC, num_subcores=NS)
    return pl.kernel(
        _sc_aggregate_body,
        out_type=jax.ShapeDtypeStruct((NC * NPAD, D), jnp.float32),
        mesh=mesh,
        scratch_types=[
            pltpu.VMEM((STAGE, CHUNK), jnp.int32),
            pltpu.VMEM((STAGE, CHUNK), jnp.int32),
            pltpu.VMEM((CHUNK, D), jnp.float32),
            pltpu.VMEM((CHUNK, D), jnp.float32),
            pltpu.VMEM((CHUNK, D), jnp.float32),
            pltpu.VMEM((CHUNK, D), jnp.float32),
            pltpu.VMEM_SHARED((NPAD, D), jnp.float32),
            pltpu.SemaphoreType.DMA((NBUF,)),
            pltpu.SemaphoreType.DMA((NBUF,)),
            pltpu.SemaphoreType.DMA,
        ],
    )


def _sc_aggregate(x, ei):
    return _build_sc_aggregate()(x, ei)


BLK = 632
GRID = (N + BLK - 1) // BLK


def _mlp_body(x_ref, p0_ref, p1_ref, w1_ref, b1_ref, w2_ref, b2_ref, o_ref):
    h = x_ref[...] + p0_ref[...] + p1_ref[...]
    h = jnp.dot(h, w1_ref[...], preferred_element_type=jnp.float32) + b1_ref[...]
    h = jnp.maximum(h, 0.0)
    h = jnp.dot(h, w2_ref[...], preferred_element_type=jnp.float32) + b2_ref[...]
    o_ref[...] = jnp.maximum(h, 0.0)


def _head_body(x_ref, p0_ref, p1_ref, w1_ref, b1_ref, w2_ref, b2_ref,
               l1w_ref, l1b_ref, l2w_ref, l2b_ref, o_ref):
    h = x_ref[...] + p0_ref[...] + p1_ref[...]
    h = jnp.dot(h, w1_ref[...], preferred_element_type=jnp.float32) + b1_ref[...]
    h = jnp.maximum(h, 0.0)
    h = jnp.dot(h, w2_ref[...], preferred_element_type=jnp.float32) + b2_ref[...]
    h = jnp.maximum(h, 0.0)
    h = jnp.dot(h, l1w_ref[...], preferred_element_type=jnp.float32) + l1b_ref[...]
    h = jnp.maximum(h, 0.0)
    l = jnp.dot(h, l2w_ref[...], preferred_element_type=jnp.float32) + l2b_ref[...]
    m = jnp.max(l, axis=-1, keepdims=True)
    lse = jnp.log(jnp.sum(jnp.exp(l - m), axis=-1, keepdims=True)) + m
    o_ref[...] = l - lse


def _row_spec():
    return pl.BlockSpec((BLK, D), lambda i: (i, 0))


def _full_spec(shape):
    return pl.BlockSpec(shape, lambda i: tuple(0 for _ in shape))


def _mlp(x, parts, w1, b1, w2, b2):
    return pl.pallas_call(
        _mlp_body,
        grid=(GRID,),
        in_specs=[
            _row_spec(),
            pl.BlockSpec((BLK, D), lambda i: (i, 0)),
            pl.BlockSpec((BLK, D), lambda i: (i + NPAD // BLK, 0)),
            _full_spec((D, D)), _full_spec((1, D)),
            _full_spec((D, D)), _full_spec((1, D)),
        ],
        out_specs=_row_spec(),
        out_shape=jax.ShapeDtypeStruct((N, D), jnp.float32),
    )(x, parts, parts, w1, b1, w2, b2)


def _head(x, parts, w1, b1, w2, b2, l1w, l1b, l2w, l2b, C):
    return pl.pallas_call(
        _head_body,
        grid=(GRID,),
        in_specs=[
            _row_spec(),
            pl.BlockSpec((BLK, D), lambda i: (i, 0)),
            pl.BlockSpec((BLK, D), lambda i: (i + NPAD // BLK, 0)),
            _full_spec((D, D)), _full_spec((1, D)),
            _full_spec((D, D)), _full_spec((1, D)),
            _full_spec((D, D)), _full_spec((1, D)),
            _full_spec((D, C)), _full_spec((1, C)),
        ],
        out_specs=pl.BlockSpec((BLK, C), lambda i: (i, 0)),
        out_shape=jax.ShapeDtypeStruct((N, C), jnp.float32),
    )(x, parts, parts, w1, b1, w2, b2, l1w, l1b, l2w, l2b)


def _fold_bn(W1, b1, g, be, m, v):
    s = g / jnp.sqrt(v + 1e-5)
    return W1 * s[None, :], ((b1 - m) * s + be)[None, :]


def kernel(x, edge_index, W1_0, b1_0, g_0, be_0, m_0, v_0, W2_0, b2_0,
           W1_1, b1_1, g_1, be_1, m_1, v_1, W2_1, b2_1,
           W1_2, b1_2, g_2, be_2, m_2, v_2, W2_2, b2_2,
           lin1_W, lin1_b, lin2_W, lin2_b):
    ei = edge_index.reshape(2, NW, NSTAGE, STAGE, CHUNK)
    C = lin2_W.shape[1]

    w1f_0, b1f_0 = _fold_bn(W1_0, b1_0, g_0, be_0, m_0, v_0)
    w1f_1, b1f_1 = _fold_bn(W1_1, b1_1, g_1, be_1, m_1, v_1)
    w1f_2, b1f_2 = _fold_bn(W1_2, b1_2, g_2, be_2, m_2, v_2)

    p = _sc_aggregate(x, ei)
    h = _mlp(x, p, w1f_0, b1f_0, W2_0, b2_0[None, :])
    p = _sc_aggregate(h, ei)
    h = _mlp(h, p, w1f_1, b1f_1, W2_1, b2_1[None, :])
    p = _sc_aggregate(h, ei)
    return _head(h, p, w1f_2, b1f_2, W2_2, b2_2[None, :],
                 lin1_W, lin1_b[None, :], lin2_W, lin2_b[None, :], C)


# speedup vs baseline: 11.4385x; 1.0866x over previous
import functools

import jax
import jax.numpy as jnp
from jax import lax
from jax.experimental import pallas as pl
from jax.experimental.pallas import tpu as pltpu
from jax.experimental.pallas import tpu_sc as plsc

N = 10000
E = 320000
D = 128

NC = 2
NS = 16
NW = NC * NS

EDGES_PER_TILE = E // NW
CHUNK = 40
NUM_CHUNKS = EDGES_PER_TILE // CHUNK
NBUF = 4
IBUF = 8
LOOK = 2
ILOOK = 6
NPAD = 10112
ROWS_PER_TILE = NPAD // NS


def _sc_aggregate_body(x_hbm, ei_hbm, out_hbm,
                       s0, s1, s2, s3, s4, s5, s6, s7,
                       d0, d1, d2, d3, d4, d5, d6, d7,
                       r0, r1, r2, r3, acc_sh, gsem, ssem, isem, zsem):
    cid = lax.axis_index("c")
    sid = lax.axis_index("s")
    wid = cid * NS + sid
    rows = [r0, r1, r2, r3]
    siq = [s0, s1, s2, s3, s4, s5, s6, s7]
    diq = [d0, d1, d2, d3, d4, d5, d6, d7]

    for r in range(CHUNK):
        for c in range(D // 16):
            r0[r, pl.ds(c * 16, 16)] = jnp.zeros((16,), jnp.float32)
    row0 = sid * ROWS_PER_TILE
    zdescs = [
        pltpu.async_copy(r0, acc_sh.at[pl.ds(row0 + j * CHUNK, CHUNK)], zsem)
        for j in range(ROWS_PER_TILE // CHUNK)
    ]
    zrem = ROWS_PER_TILE % CHUNK
    if zrem:
        zdescs.append(pltpu.async_copy(
            r0.at[pl.ds(0, zrem)],
            acc_sh.at[pl.ds(row0 + ROWS_PER_TILE - zrem, zrem)], zsem))
    for d in zdescs:
        d.wait()

    plsc.subcore_barrier()

    def i_start(c, q):
        pltpu.async_copy(ei_hbm.at[0, wid, c], siq[q], isem.at[q])
        pltpu.async_copy(ei_hbm.at[1, wid, c], diq[q], isem.at[q])

    def i_wait(q):
        pltpu.make_async_copy(ei_hbm.at[0, wid, 0], siq[q], isem.at[q]).wait()
        pltpu.make_async_copy(ei_hbm.at[1, wid, 0], diq[q], isem.at[q]).wait()

    def g_start(c, q, k):
        pltpu.async_copy(x_hbm.at[siq[q]], rows[k], gsem.at[k])

    def g_wait(k):
        pltpu.make_async_copy(x_hbm.at[siq[0]], rows[k], gsem.at[k]).wait()

    def s_start(c, q, k):
        pltpu.async_copy(rows[k], acc_sh.at[diq[q]], ssem.at[k], add=True)

    def s_wait(k):
        pltpu.make_async_copy(rows[k], acc_sh.at[diq[0]], ssem.at[k]).wait()

    def step(c, kq, kb, swait=True, istart=True, gstart=True):
        if swait:
            s_wait((kb + LOOK) % NBUF)
        if istart:
            i_start(c + ILOOK, (kq + ILOOK) % IBUF)
        if gstart:
            i_wait((kq + LOOK) % IBUF)
            g_start(c + LOOK, (kq + LOOK) % IBUF, (kb + LOOK) % NBUF)
        g_wait(kb)
        s_start(c, kq, kb)

    for c in range(ILOOK):
        i_start(c, c % IBUF)
    for c in range(LOOK):
        i_wait(c % IBUF)
        g_start(c, c % IBUF, c % NBUF)
    for c in range(IBUF):
        step(c, c % IBUF, c % NBUF, swait=(c >= LOOK))

    def octet(j):
        for k in range(IBUF):
            step(j + k, k, k % NBUF)
    pl.loop(IBUF, NUM_CHUNKS - IBUF - 2, step=IBUF)(octet)

    for c in range(NUM_CHUNKS - IBUF - 2, NUM_CHUNKS):
        step(c, c % IBUF, c % NBUF,
             istart=(c + ILOOK < NUM_CHUNKS),
             gstart=(c + LOOK < NUM_CHUNKS))
    for k in range(2):
        s_wait((NUM_CHUNKS - 2 + k) % NBUF)

    plsc.subcore_barrier()

    pltpu.sync_copy(acc_sh.at[pl.ds(row0, ROWS_PER_TILE)],
                    out_hbm.at[pl.ds(cid * NPAD + row0, ROWS_PER_TILE)])


@functools.lru_cache(maxsize=1)
def _build_sc_aggregate():
    mesh = plsc.VectorSubcoreMesh(core_axis_name="c", subcore_axis_name="s",
                                  num_cores=NC, num_subcores=NS)
    return pl.kernel(
        _sc_aggregate_body,
        out_type=jax.ShapeDtypeStruct((NC * NPAD, D), jnp.float32),
        mesh=mesh,
        scratch_types=(
            [pltpu.VMEM((CHUNK,), jnp.int32) for _ in range(2 * IBUF)]
            + [pltpu.VMEM((CHUNK, D), jnp.float32) for _ in range(NBUF)]
            + [
                pltpu.VMEM_SHARED((NPAD, D), jnp.float32),
                pltpu.SemaphoreType.DMA((NBUF,)),
                pltpu.SemaphoreType.DMA((NBUF,)),
                pltpu.SemaphoreType.DMA((IBUF,)),
                pltpu.SemaphoreType.DMA,
            ]
        ),
    )


def _sc_aggregate(x, ei):
    return _build_sc_aggregate()(x, ei)


BLK = 632
GRID = (N + BLK - 1) // BLK


def _mlp_body(x_ref, p0_ref, p1_ref, w1_ref, b1_ref, w2_ref, b2_ref, o_ref):
    h = x_ref[...] + p0_ref[...] + p1_ref[...]
    h = jnp.dot(h, w1_ref[...], preferred_element_type=jnp.float32) + b1_ref[...]
    h = jnp.maximum(h, 0.0)
    h = jnp.dot(h, w2_ref[...], preferred_element_type=jnp.float32) + b2_ref[...]
    o_ref[...] = jnp.maximum(h, 0.0)


def _head_body(x_ref, p0_ref, p1_ref, w1_ref, b1_ref, w2_ref, b2_ref,
               l1w_ref, l1b_ref, l2w_ref, l2b_ref, o_ref):
    h = x_ref[...] + p0_ref[...] + p1_ref[...]
    h = jnp.dot(h, w1_ref[...], preferred_element_type=jnp.float32) + b1_ref[...]
    h = jnp.maximum(h, 0.0)
    h = jnp.dot(h, w2_ref[...], preferred_element_type=jnp.float32) + b2_ref[...]
    h = jnp.maximum(h, 0.0)
    h = jnp.dot(h, l1w_ref[...], preferred_element_type=jnp.float32) + l1b_ref[...]
    h = jnp.maximum(h, 0.0)
    l = jnp.dot(h, l2w_ref[...], preferred_element_type=jnp.float32) + l2b_ref[...]
    m = jnp.max(l, axis=-1, keepdims=True)
    lse = jnp.log(jnp.sum(jnp.exp(l - m), axis=-1, keepdims=True)) + m
    o_ref[...] = l - lse


def _row_spec():
    return pl.BlockSpec((BLK, D), lambda i: (i, 0))


def _full_spec(shape):
    return pl.BlockSpec(shape, lambda i: tuple(0 for _ in shape))


def _mlp(x, parts, w1, b1, w2, b2):
    return pl.pallas_call(
        _mlp_body,
        grid=(GRID,),
        in_specs=[
            _row_spec(),
            pl.BlockSpec((BLK, D), lambda i: (i, 0)),
            pl.BlockSpec((BLK, D), lambda i: (i + NPAD // BLK, 0)),
            _full_spec((D, D)), _full_spec((1, D)),
            _full_spec((D, D)), _full_spec((1, D)),
        ],
        out_specs=_row_spec(),
        out_shape=jax.ShapeDtypeStruct((N, D), jnp.float32),
    )(x, parts, parts, w1, b1, w2, b2)


def _head(x, parts, w1, b1, w2, b2, l1w, l1b, l2w, l2b, C):
    return pl.pallas_call(
        _head_body,
        grid=(GRID,),
        in_specs=[
            _row_spec(),
            pl.BlockSpec((BLK, D), lambda i: (i, 0)),
            pl.BlockSpec((BLK, D), lambda i: (i + NPAD // BLK, 0)),
            _full_spec((D, D)), _full_spec((1, D)),
            _full_spec((D, D)), _full_spec((1, D)),
            _full_spec((D, D)), _full_spec((1, D)),
            _full_spec((D, C)), _full_spec((1, C)),
        ],
        out_specs=pl.BlockSpec((BLK, C), lambda i: (i, 0)),
        out_shape=jax.ShapeDtypeStruct((N, C), jnp.float32),
    )(x, parts, parts, w1, b1, w2, b2, l1w, l1b, l2w, l2b)


def _fold_bn(W1, b1, g, be, m, v):
    s = g / jnp.sqrt(v + 1e-5)
    return W1 * s[None, :], ((b1 - m) * s + be)[None, :]


def kernel(x, edge_index, W1_0, b1_0, g_0, be_0, m_0, v_0, W2_0, b2_0,
           W1_1, b1_1, g_1, be_1, m_1, v_1, W2_1, b2_1,
           W1_2, b1_2, g_2, be_2, m_2, v_2, W2_2, b2_2,
           lin1_W, lin1_b, lin2_W, lin2_b):
    ei = edge_index.reshape(2, NW, NUM_CHUNKS, CHUNK)
    C = lin2_W.shape[1]

    w1f_0, b1f_0 = _fold_bn(W1_0, b1_0, g_0, be_0, m_0, v_0)
    w1f_1, b1f_1 = _fold_bn(W1_1, b1_1, g_1, be_1, m_1, v_1)
    w1f_2, b1f_2 = _fold_bn(W1_2, b1_2, g_2, be_2, m_2, v_2)

    p = _sc_aggregate(x, ei)
    h = _mlp(x, p, w1f_0, b1f_0, W2_0, b2_0[None, :])
    p = _sc_aggregate(h, ei)
    h = _mlp(h, p, w1f_1, b1f_1, W2_1, b2_1[None, :])
    p = _sc_aggregate(h, ei)
    return _head(h, p, w1f_2, b1f_2, W2_2, b2_2[None, :],
                 lin1_W, lin1_b[None, :], lin2_W, lin2_b[None, :], C)


# speedup vs baseline: 11.5451x; 1.0093x over previous
import functools

import jax
import jax.numpy as jnp
from jax import lax
from jax.experimental import pallas as pl
from jax.experimental.pallas import tpu as pltpu
from jax.experimental.pallas import tpu_sc as plsc

N = 10000
E = 320000
D = 128

NC = 2
NS = 16
NW = NC * NS

EDGES_PER_TILE = E // NW
CHUNK = 40
NUM_CHUNKS = EDGES_PER_TILE // CHUNK
NBUF = 4
IBUF = 8
LOOK = 2
ILOOK = 6
NPAD = 10112
ROWS_PER_TILE = NPAD // NS


def _sc_aggregate_body(x_hbm, ei_hbm, out_hbm,
                       s0, s1, s2, s3, s4, s5, s6, s7,
                       d0, d1, d2, d3, d4, d5, d6, d7,
                       r0, r1, r2, r3, acc_sh, gsem, ssem, isem, zsem):
    cid = lax.axis_index("c")
    sid = lax.axis_index("s")
    wid = cid * NS + sid
    rows = [r0, r1, r2, r3]
    siq = [s0, s1, s2, s3, s4, s5, s6, s7]
    diq = [d0, d1, d2, d3, d4, d5, d6, d7]

    for r in range(CHUNK):
        for c in range(D // 16):
            r3[r, pl.ds(c * 16, 16)] = jnp.zeros((16,), jnp.float32)
    row0 = sid * ROWS_PER_TILE
    zdescs = [
        pltpu.async_copy(r3, acc_sh.at[pl.ds(row0 + j * CHUNK, CHUNK)], zsem)
        for j in range(ROWS_PER_TILE // CHUNK)
    ]
    zrem = ROWS_PER_TILE % CHUNK
    if zrem:
        zdescs.append(pltpu.async_copy(
            r3.at[pl.ds(0, zrem)],
            acc_sh.at[pl.ds(row0 + ROWS_PER_TILE - zrem, zrem)], zsem))

    def i_start(c, q):
        pltpu.async_copy(ei_hbm.at[0, wid, c], siq[q], isem.at[q])
        pltpu.async_copy(ei_hbm.at[1, wid, c], diq[q], isem.at[q])

    def i_wait(q):
        pltpu.make_async_copy(ei_hbm.at[0, wid, 0], siq[q], isem.at[q]).wait()
        pltpu.make_async_copy(ei_hbm.at[1, wid, 0], diq[q], isem.at[q]).wait()

    def g_start(c, q, k):
        pltpu.async_copy(x_hbm.at[siq[q]], rows[k], gsem.at[k])

    def g_wait(k):
        pltpu.make_async_copy(x_hbm.at[siq[0]], rows[k], gsem.at[k]).wait()

    def s_start(c, q, k):
        pltpu.async_copy(rows[k], acc_sh.at[diq[q]], ssem.at[k], add=True)

    def s_wait(k):
        pltpu.make_async_copy(rows[k], acc_sh.at[diq[0]], ssem.at[k]).wait()

    def step(c, kq, kb, swait=True, istart=True, gstart=True):
        if swait:
            s_wait((kb + LOOK) % NBUF)
        if istart:
            i_start(c + ILOOK, (kq + ILOOK) % IBUF)
        if gstart:
            i_wait((kq + LOOK) % IBUF)
            g_start(c + LOOK, (kq + LOOK) % IBUF, (kb + LOOK) % NBUF)
        g_wait(kb)
        s_start(c, kq, kb)

    for c in range(ILOOK):
        i_start(c, c % IBUF)
    for c in range(LOOK):
        i_wait(c % IBUF)
        g_start(c, c % IBUF, c % NBUF)
    for d in zdescs:
        d.wait()
    plsc.subcore_barrier()
    for c in range(IBUF):
        step(c, c % IBUF, c % NBUF, swait=(c >= LOOK))

    def octet(j):
        for k in range(IBUF):
            step(j + k, k, k % NBUF)
    pl.loop(IBUF, NUM_CHUNKS - IBUF - 2, step=IBUF)(octet)

    for c in range(NUM_CHUNKS - IBUF - 2, NUM_CHUNKS):
        step(c, c % IBUF, c % NBUF,
             istart=(c + ILOOK < NUM_CHUNKS),
             gstart=(c + LOOK < NUM_CHUNKS))
    for k in range(2):
        s_wait((NUM_CHUNKS - 2 + k) % NBUF)

    plsc.subcore_barrier()

    pltpu.sync_copy(acc_sh.at[pl.ds(row0, ROWS_PER_TILE)],
                    out_hbm.at[pl.ds(cid * NPAD + row0, ROWS_PER_TILE)])


@functools.lru_cache(maxsize=1)
def _build_sc_aggregate():
    mesh = plsc.VectorSubcoreMesh(core_axis_name="c", subcore_axis_name="s",
                                  num_cores=NC, num_subcores=NS)
    return pl.kernel(
        _sc_aggregate_body,
        out_type=jax.ShapeDtypeStruct((NC * NPAD, D), jnp.float32),
        mesh=mesh,
        scratch_types=(
            [pltpu.VMEM((CHUNK,), jnp.int32) for _ in range(2 * IBUF)]
            + [pltpu.VMEM((CHUNK, D), jnp.float32) for _ in range(NBUF)]
            + [
                pltpu.VMEM_SHARED((NPAD, D), jnp.float32),
                pltpu.SemaphoreType.DMA((NBUF,)),
                pltpu.SemaphoreType.DMA((NBUF,)),
                pltpu.SemaphoreType.DMA((IBUF,)),
                pltpu.SemaphoreType.DMA,
            ]
        ),
    )


def _sc_aggregate(x, ei):
    return _build_sc_aggregate()(x, ei)


BLK = 632
GRID = (N + BLK - 1) // BLK


def _mlp_body(x_ref, p0_ref, p1_ref, w1_ref, b1_ref, w2_ref, b2_ref, o_ref):
    h = x_ref[...] + p0_ref[...] + p1_ref[...]
    h = jnp.dot(h, w1_ref[...], preferred_element_type=jnp.float32,
                precision=lax.Precision.DEFAULT) + b1_ref[...]
    h = jnp.maximum(h, 0.0)
    h = jnp.dot(h, w2_ref[...], preferred_element_type=jnp.float32,
                precision=lax.Precision.DEFAULT) + b2_ref[...]
    o_ref[...] = jnp.maximum(h, 0.0)


def _head_body(x_ref, p0_ref, p1_ref, w1_ref, b1_ref, w2_ref, b2_ref,
               l1w_ref, l1b_ref, l2w_ref, l2b_ref, o_ref):
    h = x_ref[...] + p0_ref[...] + p1_ref[...]
    h = jnp.dot(h, w1_ref[...], preferred_element_type=jnp.float32,
                precision=lax.Precision.DEFAULT) + b1_ref[...]
    h = jnp.maximum(h, 0.0)
    h = jnp.dot(h, w2_ref[...], preferred_element_type=jnp.float32,
                precision=lax.Precision.DEFAULT) + b2_ref[...]
    h = jnp.maximum(h, 0.0)
    h = jnp.dot(h, l1w_ref[...], preferred_element_type=jnp.float32,
                precision=lax.Precision.DEFAULT) + l1b_ref[...]
    h = jnp.maximum(h, 0.0)
    l = jnp.dot(h, l2w_ref[...], preferred_element_type=jnp.float32,
                precision=lax.Precision.DEFAULT) + l2b_ref[...]
    m = jnp.max(l, axis=-1, keepdims=True)
    lse = jnp.log(jnp.sum(jnp.exp(l - m), axis=-1, keepdims=True)) + m
    o_ref[...] = l - lse


def _row_spec():
    return pl.BlockSpec((BLK, D), lambda i: (i, 0))


def _full_spec(shape):
    return pl.BlockSpec(shape, lambda i: tuple(0 for _ in shape))


def _mlp(x, parts, w1, b1, w2, b2):
    return pl.pallas_call(
        _mlp_body,
        grid=(GRID,),
        in_specs=[
            _row_spec(),
            pl.BlockSpec((BLK, D), lambda i: (i, 0)),
            pl.BlockSpec((BLK, D), lambda i: (i + NPAD // BLK, 0)),
            _full_spec((D, D)), _full_spec((1, D)),
            _full_spec((D, D)), _full_spec((1, D)),
        ],
        out_specs=_row_spec(),
        out_shape=jax.ShapeDtypeStruct((N, D), jnp.float32),
    )(x, parts, parts, w1, b1, w2, b2)


def _head(x, parts, w1, b1, w2, b2, l1w, l1b, l2w, l2b, C):
    return pl.pallas_call(
        _head_body,
        grid=(GRID,),
        in_specs=[
            _row_spec(),
            pl.BlockSpec((BLK, D), lambda i: (i, 0)),
            pl.BlockSpec((BLK, D), lambda i: (i + NPAD // BLK, 0)),
            _full_spec((D, D)), _full_spec((1, D)),
            _full_spec((D, D)), _full_spec((1, D)),
            _full_spec((D, D)), _full_spec((1, D)),
            _full_spec((D, C)), _full_spec((1, C)),
        ],
        out_specs=pl.BlockSpec((BLK, C), lambda i: (i, 0)),
        out_shape=jax.ShapeDtypeStruct((N, C), jnp.float32),
    )(x, parts, parts, w1, b1, w2, b2, l1w, l1b, l2w, l2b)


def _fold_bn(W1, b1, g, be, m, v):
    s = g / jnp.sqrt(v + 1e-5)
    return W1 * s[None, :], ((b1 - m) * s + be)[None, :]


def kernel(x, edge_index, W1_0, b1_0, g_0, be_0, m_0, v_0, W2_0, b2_0,
           W1_1, b1_1, g_1, be_1, m_1, v_1, W2_1, b2_1,
           W1_2, b1_2, g_2, be_2, m_2, v_2, W2_2, b2_2,
           lin1_W, lin1_b, lin2_W, lin2_b):
    ei = edge_index.reshape(2, NW, NUM_CHUNKS, CHUNK)
    C = lin2_W.shape[1]

    w1f_0, b1f_0 = _fold_bn(W1_0, b1_0, g_0, be_0, m_0, v_0)
    w1f_1, b1f_1 = _fold_bn(W1_1, b1_1, g_1, be_1, m_1, v_1)
    w1f_2, b1f_2 = _fold_bn(W1_2, b1_2, g_2, be_2, m_2, v_2)

    p = _sc_aggregate(x, ei)
    h = _mlp(x, p, w1f_0, b1f_0, W2_0, b2_0[None, :])
    p = _sc_aggregate(h, ei)
    h = _mlp(h, p, w1f_1, b1f_1, W2_1, b2_1[None, :])
    p = _sc_aggregate(h, ei)
    return _head(h, p, w1f_2, b1f_2, W2_2, b2_2[None, :],
                 lin1_W, lin1_b[None, :], lin2_W, lin2_b[None, :], C)
